# Initial kernel scaffold; baseline (speedup 1.0000x reference)
import jax
import jax.numpy as jnp
from jax.experimental import pallas as pl


def kernel(reward, feat, proj, queue):
    raise NotImplementedError("write your pallas kernel here")


# speedup vs baseline: 7.5417x; 7.5417x over previous
import functools

import jax
import jax.numpy as jnp
from jax.experimental import pallas as pl

_B = 16
_T = 66
_L = 3
_F = 1024
_D = 64
_K = 16
_TT = _T - _L + 1
_N = _B * _TT
_MOMENTUM = 0.99
_EPS = 1e-8
_BETA = 1.0
_SCALE = 1.0


def _knn_kernel(feat_ref, proj_ref, rew_ref, out_ref):
    f = feat_ref[...]
    w = (f[:, 0:_TT, :] + f[:, 1:_TT + 1, :] + f[:, 2:_TT + 2, :]) * (1.0 / _L)
    w2 = w.reshape(_N, _F)

    sf = jnp.dot(w2, proj_ref[...], preferred_element_type=jnp.float32)

    g = jax.lax.dot_general(sf, sf, (((1,), (1,)), ((), ())),
                            preferred_element_type=jnp.float32)
    n2 = jnp.sum(sf * sf, axis=1, keepdims=True)
    d2 = jnp.maximum(n2 + n2.reshape(1, _N) - 2.0 * g, 0.0)
    dist = jnp.sqrt(d2)

    iota = jax.lax.broadcasted_iota(jnp.int32, (_N, _N), 1)
    vals = dist
    total = jnp.zeros((_N, 1), dtype=jnp.float32)
    for _ in range(_K):
        m = jnp.min(vals, axis=1, keepdims=True)
        total = total + m
        first = jnp.min(jnp.where(vals <= m, iota, _N), axis=1, keepdims=True)
        vals = jnp.where(iota == first, jnp.inf, vals)
    int_rew = total * (1.0 / _K)

    mag = _MOMENTUM + (1.0 - _MOMENTUM) * jnp.mean(jnp.abs(int_rew))
    out_ref[...] = rew_ref[...] + _BETA * _SCALE * int_rew / (mag + _EPS)


@jax.jit
def kernel(reward, feat, proj, queue):
    del queue
    rew2 = reward[:, :_TT].reshape(_N, 1)
    out = pl.pallas_call(
        _knn_kernel,
        out_shape=jax.ShapeDtypeStruct((_N, 1), jnp.float32),
    )(feat, proj, rew2)
    return out.reshape(_B, _TT, 1)


# speedup vs baseline: 8.1544x; 1.0812x over previous
import functools

import jax
import jax.numpy as jnp
from jax.experimental import pallas as pl

_B = 16
_T = 66
_L = 3
_F = 1024
_D = 64
_K = 16
_TT = _T - _L + 1
_N = _B * _TT
_MOMENTUM = 0.99
_EPS = 1e-8
_BETA = 1.0
_SCALE = 1.0


def _knn_kernel(feat_ref, proj_ref, rew_ref, out_ref):
    f = feat_ref[...]
    w = (f[:, 0:_TT, :] + f[:, 1:_TT + 1, :] + f[:, 2:_TT + 2, :]) * (1.0 / _L)
    w2 = w.reshape(_N, _F)

    sf = jnp.dot(w2, proj_ref[...], preferred_element_type=jnp.float32)

    g = jax.lax.dot_general(sf, sf, (((1,), (1,)), ((), ())),
                            preferred_element_type=jnp.float32)
    n2 = jnp.sum(sf * sf, axis=1, keepdims=True)
    d2 = jnp.maximum(n2 + n2.reshape(1, _N) - 2.0 * g, 0.0)

    vals = d2
    total = jnp.zeros((_N, 1), dtype=jnp.float32)
    remaining = jnp.full((_N, 1), float(_K), dtype=jnp.float32)
    for _ in range(_K):
        m = jnp.min(vals, axis=1, keepdims=True)
        hit = vals <= m
        cnt = jnp.sum(hit.astype(jnp.float32), axis=1, keepdims=True)
        take = jnp.minimum(cnt, jnp.maximum(remaining, 0.0))
        total = total + take * jnp.sqrt(m)
        remaining = remaining - cnt
        vals = jnp.where(hit, jnp.inf, vals)
    int_rew = total * (1.0 / _K)

    mag = _MOMENTUM + (1.0 - _MOMENTUM) * jnp.mean(jnp.abs(int_rew))
    out_ref[...] = rew_ref[...] + _BETA * _SCALE * int_rew / (mag + _EPS)


@jax.jit
def kernel(reward, feat, proj, queue):
    del queue
    rew2 = reward[:, :_TT].reshape(_N, 1)
    out = pl.pallas_call(
        _knn_kernel,
        out_shape=jax.ShapeDtypeStruct((_N, 1), jnp.float32),
    )(feat, proj, rew2)
    return out.reshape(_B, _TT, 1)
